# baseline (speedup 1.0000x reference)
import functools

import jax
import jax.numpy as jnp
from jax import lax
from jax.experimental import pallas as pl
from jax.experimental.pallas import tpu as pltpu
from jax.experimental.pallas import tpu_sc as plsc

N_NODES = 10000
N_EDGES = 320000
D_FEAT = 128

NC = 2
NS = 16
L = 16
NW = NC * NS

PER_W = N_EDGES // NW
C = 80
N_CHUNKS = PER_W // C
G = C // L
ROWS_PER_TILE = (N_NODES // NS) // 8 * 8


def _tile_body(src_hbm, dst_hbm, feats_hbm, out_hbm,
               idx_s0, idx_d0, idx_s1, idx_d1, out0, out1,
               rows_s0, rows_d0, rows_s1, rows_d1, tr, feats_sp,
               sem_i0, sem_i1, sem_g0, sem_g1, sem_o0, sem_o1):
  wid = lax.axis_index("s") * NC + lax.axis_index("c")
  sid = lax.axis_index("s")
  iota = lax.iota(jnp.int32, L)
  base_w = wid * PER_W

  row0 = sid * ROWS_PER_TILE
  pltpu.sync_copy(feats_hbm.at[pl.ds(row0, ROWS_PER_TILE)],
                  feats_sp.at[pl.ds(row0, ROWS_PER_TILE)])
  tail = NS * ROWS_PER_TILE

  @pl.when(sid == NS - 1)
  def _copy_tail():
    pltpu.sync_copy(feats_hbm.at[pl.ds(tail, N_NODES - tail)],
                    feats_sp.at[pl.ds(tail, N_NODES - tail)])

  plsc.subcore_barrier()

  def start_idx(k, bis, bid, sem):
    pltpu.async_copy(src_hbm.at[pl.ds(base_w + k * C, C)], bis, sem)
    pltpu.async_copy(dst_hbm.at[pl.ds(base_w + k * C, C)], bid, sem)

  def wait_idx(k, bis, bid, sem):
    pltpu.make_async_copy(src_hbm.at[pl.ds(base_w + k * C, C)], bis,
                          sem).wait()
    pltpu.make_async_copy(dst_hbm.at[pl.ds(base_w + k * C, C)], bid,
                          sem).wait()

  def start_gather(bis, bid, bs, bd, sem):
    pltpu.async_copy(feats_sp.at[bis], bs, sem)
    pltpu.async_copy(feats_sp.at[bid], bd, sem)

  def wait_gather(bis, bid, bs, bd, sem):
    pltpu.make_async_copy(feats_sp.at[bis], bs, sem).wait()
    pltpu.make_async_copy(feats_sp.at[bid], bd, sem).wait()

  def start_out(k, bo, sem):
    pltpu.async_copy(bo, out_hbm.at[pl.ds(base_w + k * C, C)], sem)

  def wait_out(k, bo, sem):
    pltpu.make_async_copy(bo, out_hbm.at[pl.ds(base_w + k * C, C)],
                          sem).wait()

  iota17 = iota * 17

  def compute(bs, bd, bo):
    def group(g, _):
      for e in range(L):
        acc = jnp.zeros((L,), jnp.float32)
        for j in range(D_FEAT // L):
          sv = bs[g * L + e, pl.ds(j * L, L)]
          dv = bd[g * L + e, pl.ds(j * L, L)]
          acc = acc + sv * dv
        tr[pl.ds(e * 17, L)] = acc
      res = jnp.zeros((L,), jnp.float32)
      for c in range(L):
        res = res + plsc.load_gather(tr, [iota17 + c])
      bo[pl.ds(g * L, L)] = 1.0 / (1.0 + jnp.exp(-res))
      return ()

    lax.fori_loop(0, G, group, ())

  start_idx(0, idx_s0, idx_d0, sem_i0)
  wait_idx(0, idx_s0, idx_d0, sem_i0)
  start_gather(idx_s0, idx_d0, rows_s0, rows_d0, sem_g0)
  start_idx(1, idx_s1, idx_d1, sem_i1)

  def pair(i, _):
    e, o, n = 2 * i, 2 * i + 1, 2 * i + 2
    n_odd = jnp.minimum(n + 1, N_CHUNKS - 1)
    wait_idx(o, idx_s1, idx_d1, sem_i1)
    start_gather(idx_s1, idx_d1, rows_s1, rows_d1, sem_g1)
    wait_gather(idx_s0, idx_d0, rows_s0, rows_d0, sem_g0)
    wait_out(e, out0, sem_o0)
    compute(rows_s0, rows_d0, out0)
    start_out(e, out0, sem_o0)
    start_idx(n, idx_s0, idx_d0, sem_i0)
    wait_idx(n, idx_s0, idx_d0, sem_i0)
    start_gather(idx_s0, idx_d0, rows_s0, rows_d0, sem_g0)
    wait_gather(idx_s1, idx_d1, rows_s1, rows_d1, sem_g1)
    start_idx(n_odd, idx_s1, idx_d1, sem_i1)
    wait_out(o, out1, sem_o1)
    compute(rows_s1, rows_d1, out1)
    start_out(o, out1, sem_o1)
    return ()

  start_out(0, out0, sem_o0)
  start_out(1, out1, sem_o1)

  lax.fori_loop(0, (N_CHUNKS - 1) // 2, pair, ())
  last = N_CHUNKS - 1
  wait_gather(idx_s0, idx_d0, rows_s0, rows_d0, sem_g0)
  wait_out(last, out0, sem_o0)
  compute(rows_s0, rows_d0, out0)
  start_out(last, out0, sem_o0)
  wait_out(last, out0, sem_o0)
  wait_out(last - 1, out1, sem_o1)
  wait_idx(last, idx_s1, idx_d1, sem_i1)


def kernel(src_list, dst_list, feats):
  mesh = plsc.VectorSubcoreMesh(core_axis_name="c", subcore_axis_name="s")
  run = functools.partial(
      pl.kernel,
      out_type=jax.ShapeDtypeStruct((N_EDGES,), jnp.float32),
      mesh=mesh,
      compiler_params=pltpu.CompilerParams(needs_layout_passes=False),
      scratch_types=[
          pltpu.VMEM((C,), jnp.int32),
          pltpu.VMEM((C,), jnp.int32),
          pltpu.VMEM((C,), jnp.int32),
          pltpu.VMEM((C,), jnp.int32),
          pltpu.VMEM((C,), jnp.float32),
          pltpu.VMEM((C,), jnp.float32),
          pltpu.VMEM((C, D_FEAT), jnp.float32),
          pltpu.VMEM((C, D_FEAT), jnp.float32),
          pltpu.VMEM((C, D_FEAT), jnp.float32),
          pltpu.VMEM((C, D_FEAT), jnp.float32),
          pltpu.VMEM((L * 17,), jnp.float32),
          pltpu.VMEM_SHARED((N_NODES, D_FEAT), jnp.float32),
          pltpu.SemaphoreType.DMA,
          pltpu.SemaphoreType.DMA,
          pltpu.SemaphoreType.DMA,
          pltpu.SemaphoreType.DMA,
          pltpu.SemaphoreType.DMA,
          pltpu.SemaphoreType.DMA,
      ],
  )(_tile_body)
  return run(src_list, dst_list, feats)


# speedup vs baseline: 1.1517x; 1.1517x over previous
import functools

import jax
import jax.numpy as jnp
from jax import lax
from jax.experimental import pallas as pl
from jax.experimental.pallas import tpu as pltpu
from jax.experimental.pallas import tpu_sc as plsc

N_NODES = 10000
N_EDGES = 320000
D_FEAT = 128

NC = 2
NS = 16
L = 16
NW = NC * NS

PER_W = N_EDGES // NW
C = 80
N_CHUNKS = PER_W // C
G = C // L


def _tile_body(src_hbm, dst_hbm, feats_hbm, out_hbm,
               idx_s_all, idx_d_all, out_all,
               rows_s0, rows_d0, rows_s1, rows_d1, rows_s2, rows_d2, tr,
               sem0, sem1, sem2):
  wid = lax.axis_index("s") * NC + lax.axis_index("c")
  iota = lax.iota(jnp.int32, L)
  base_w = wid * PER_W

  pltpu.sync_copy(src_hbm.at[pl.ds(base_w, PER_W)], idx_s_all)
  pltpu.sync_copy(dst_hbm.at[pl.ds(base_w, PER_W)], idx_d_all)

  def start(k, bs, bd, sem):
    pltpu.async_copy(feats_hbm.at[idx_s_all.at[pl.ds(k * C, C)]], bs, sem)
    pltpu.async_copy(feats_hbm.at[idx_d_all.at[pl.ds(k * C, C)]], bd, sem)

  def wait(k, bs, bd, sem):
    pltpu.make_async_copy(feats_hbm.at[idx_s_all.at[pl.ds(k * C, C)]], bs,
                          sem).wait()
    pltpu.make_async_copy(feats_hbm.at[idx_d_all.at[pl.ds(k * C, C)]], bd,
                          sem).wait()

  iota17 = iota * 17

  def compute(k, bs, bd):
    def group(g, _):
      for e in range(L):
        acc = jnp.zeros((L,), jnp.float32)
        for j in range(D_FEAT // L):
          sv = bs[g * L + e, pl.ds(j * L, L)]
          dv = bd[g * L + e, pl.ds(j * L, L)]
          acc = acc + sv * dv
        tr[pl.ds(e * 17, L)] = acc
      res = jnp.zeros((L,), jnp.float32)
      for c in range(L):
        res = res + plsc.load_gather(tr, [iota17 + c])
      out_all[pl.ds(k * C + g * L, L)] = 1.0 / (1.0 + jnp.exp(-res))
      return ()

    lax.fori_loop(0, G, group, ())

  start(0, rows_s0, rows_d0, sem0)
  start(1, rows_s1, rows_d1, sem1)

  def triple(i, _):
    t = 3 * i
    start(t + 2, rows_s2, rows_d2, sem2)
    wait(t, rows_s0, rows_d0, sem0)
    compute(t, rows_s0, rows_d0)
    start(t + 3, rows_s0, rows_d0, sem0)
    wait(t + 1, rows_s1, rows_d1, sem1)
    compute(t + 1, rows_s1, rows_d1)
    start(t + 4, rows_s1, rows_d1, sem1)
    wait(t + 2, rows_s2, rows_d2, sem2)
    compute(t + 2, rows_s2, rows_d2)
    return ()

  lax.fori_loop(0, (N_CHUNKS - 2) // 3, triple, ())
  wait(N_CHUNKS - 2, rows_s0, rows_d0, sem0)
  compute(N_CHUNKS - 2, rows_s0, rows_d0)
  wait(N_CHUNKS - 1, rows_s1, rows_d1, sem1)
  compute(N_CHUNKS - 1, rows_s1, rows_d1)

  pltpu.sync_copy(out_all, out_hbm.at[pl.ds(base_w, PER_W)])


def kernel(src_list, dst_list, feats):
  mesh = plsc.VectorSubcoreMesh(core_axis_name="c", subcore_axis_name="s")
  run = functools.partial(
      pl.kernel,
      out_type=jax.ShapeDtypeStruct((N_EDGES,), jnp.float32),
      mesh=mesh,
      compiler_params=pltpu.CompilerParams(needs_layout_passes=False),
      scratch_types=[
          pltpu.VMEM((PER_W,), jnp.int32),
          pltpu.VMEM((PER_W,), jnp.int32),
          pltpu.VMEM((PER_W,), jnp.float32),
          pltpu.VMEM((C, D_FEAT), jnp.float32),
          pltpu.VMEM((C, D_FEAT), jnp.float32),
          pltpu.VMEM((C, D_FEAT), jnp.float32),
          pltpu.VMEM((C, D_FEAT), jnp.float32),
          pltpu.VMEM((C, D_FEAT), jnp.float32),
          pltpu.VMEM((C, D_FEAT), jnp.float32),
          pltpu.VMEM((L * 17,), jnp.float32),
          pltpu.SemaphoreType.DMA,
          pltpu.SemaphoreType.DMA,
          pltpu.SemaphoreType.DMA,
      ],
  )(_tile_body)
  return run(src_list, dst_list, feats)
